# Initial kernel scaffold; baseline (speedup 1.0000x reference)
import jax
import jax.numpy as jnp
from jax.experimental import pallas as pl


def kernel(x_user, x_item, x_tag, params, ei_ui, ei_iu, ei_it, ei_ti):
    raise NotImplementedError("write your pallas kernel here")


# speedup vs baseline: 24.5366x; 24.5366x over previous
import dataclasses
import functools

import jax
import jax.numpy as jnp
from jax import lax
from jax.experimental import pallas as pl
from jax.experimental.pallas import tpu as pltpu
from jax.experimental.pallas import tpu_sc as plsc

F32 = jnp.float32
I32 = jnp.int32

C = 128
HH = 4
DD = C // HH
NNODE = 10000
NPAD = 10240
BR = 256
GRID = NPAD // BR
EK = 128
NW = 32
NSUB = 16
NACC = 10048
NSLICE = NACC // NSUB
AUGW = 144

_NT = ('user', 'item', 'tag')
_EM = {'ui': ('user', 'item'), 'iu': ('item', 'user'),
       'it': ('item', 'tag'), 'ti': ('tag', 'item')}


def _proj_body(x_ref, w_ref, b_ref, amat_ref, h_ref, aux_ref, mx_ref):
    x = x_ref[...]
    h = jnp.dot(x, w_ref[...], preferred_element_type=F32) + b_ref[...]
    aux = jnp.dot(h, amat_ref[...], preferred_element_type=F32)
    h_ref[...] = h
    aux_ref[...] = aux
    bm = jnp.broadcast_to(jnp.max(aux, axis=0, keepdims=True), (8, 16))
    pid = pl.program_id(0)

    @pl.when(pid == 0)
    def _():
        mx_ref[...] = bm

    @pl.when(pid != 0)
    def _():
        mx_ref[...] = jnp.maximum(mx_ref[...], bm)


def _proj2_body(r0_ref, r1_ref, attn_ref, w_ref, b_ref, amat_ref,
                h_ref, aux_ref, mx_ref):
    x = attn_ref[0:1, :] * r0_ref[...] + attn_ref[1:2, :] * r1_ref[...]
    h = jnp.dot(x, w_ref[...], preferred_element_type=F32) + b_ref[...]
    aux = jnp.dot(h, amat_ref[...], preferred_element_type=F32)
    h_ref[...] = h
    aux_ref[...] = aux
    bm = jnp.broadcast_to(jnp.max(aux, axis=0, keepdims=True), (8, 16))
    pid = pl.program_id(0)

    @pl.when(pid == 0)
    def _():
        mx_ref[...] = bm

    @pl.when(pid != 0)
    def _():
        mx_ref[...] = jnp.maximum(mx_ref[...], bm)


_PROJ_OUT = (
    jax.ShapeDtypeStruct((NPAD, C), F32),
    jax.ShapeDtypeStruct((NPAD, 16), F32),
    jax.ShapeDtypeStruct((8, 16), F32),
)
_PROJ_OUT_SPECS = (
    pl.BlockSpec((BR, C), lambda i: (i, 0)),
    pl.BlockSpec((BR, 16), lambda i: (i, 0)),
    pl.BlockSpec((8, 16), lambda i: (0, 0)),
)


def _proj1(x, w, b, amat):
    return pl.pallas_call(
        _proj_body,
        grid=(GRID,),
        in_specs=[
            pl.BlockSpec((BR, C), lambda i: (i, 0)),
            pl.BlockSpec((C, C), lambda i: (0, 0)),
            pl.BlockSpec((1, C), lambda i: (0, 0)),
            pl.BlockSpec((C, 16), lambda i: (0, 0)),
        ],
        out_specs=_PROJ_OUT_SPECS,
        out_shape=_PROJ_OUT,
    )(x, w, b, amat)


def _proj2(r0, r1, attn, w, b, amat):
    return pl.pallas_call(
        _proj2_body,
        grid=(GRID,),
        in_specs=[
            pl.BlockSpec((BR, C), lambda i: (i, 0)),
            pl.BlockSpec((BR, C), lambda i: (i, 0)),
            pl.BlockSpec((2, C), lambda i: (0, 0)),
            pl.BlockSpec((C, C), lambda i: (0, 0)),
            pl.BlockSpec((1, C), lambda i: (0, 0)),
            pl.BlockSpec((C, 16), lambda i: (0, 0)),
        ],
        out_specs=_PROJ_OUT_SPECS,
        out_shape=_PROJ_OUT,
    )(r0, r1, attn, w, b, amat)


def _norm_relu(part_ref):
    p = part_ref[0] + part_ref[1]
    unnorm = p[:, :C]
    den8 = p[:, C:C + 8]
    i0 = lax.broadcasted_iota(I32, (8, C), 0)
    i1 = lax.broadcasted_iota(I32, (8, C), 1)
    p8 = (i1 // DD == i0).astype(F32)
    drep = jnp.dot(den8, p8, preferred_element_type=F32)
    r = jnp.where(drep > 0, unnorm / jnp.maximum(drep, 1e-30), 0.0)
    return jnp.maximum(r, 0.0)


def _post_simple_body(part_ref, r_ref):
    r_ref[...] = _norm_relu(part_ref)


def _post_scored_body(part_ref, kw_ref, kb_ref, q_ref, r_ref, sc_ref):
    r = _norm_relu(part_ref)
    r_ref[...] = r
    t = jnp.tanh(jnp.dot(r, kw_ref[...], preferred_element_type=F32)
                 + kb_ref[...])
    pid = pl.program_id(0)
    row = pid * BR + lax.broadcasted_iota(I32, (BR, C), 0)
    tm = jnp.where(row < NNODE, t, 0.0)
    s = jnp.sum(tm * q_ref[...])
    sb = jnp.full((8, 128), s, F32)

    @pl.when(pid == 0)
    def _():
        sc_ref[...] = sb

    @pl.when(pid != 0)
    def _():
        sc_ref[...] = sc_ref[...] + sb


_PART_SPEC = pl.BlockSpec((2, BR, AUGW), lambda i: (0, i, 0))


def _post_simple(part):
    return pl.pallas_call(
        _post_simple_body,
        grid=(GRID,),
        in_specs=[_PART_SPEC],
        out_specs=pl.BlockSpec((BR, C), lambda i: (i, 0)),
        out_shape=jax.ShapeDtypeStruct((NPAD, C), F32),
    )(part)


def _post_scored(part, kw, kb, q):
    return pl.pallas_call(
        _post_scored_body,
        grid=(GRID,),
        in_specs=[
            _PART_SPEC,
            pl.BlockSpec((C, C), lambda i: (0, 0)),
            pl.BlockSpec((1, C), lambda i: (0, 0)),
            pl.BlockSpec((1, C), lambda i: (0, 0)),
        ],
        out_specs=(
            pl.BlockSpec((BR, C), lambda i: (i, 0)),
            pl.BlockSpec((8, 128), lambda i: (0, 0)),
        ),
        out_shape=(
            jax.ShapeDtypeStruct((NPAD, C), F32),
            jax.ShapeDtypeStruct((8, 128), F32),
        ),
    )(part, kw, kb, q)


def _combine_body(r0_ref, r1_ref, attn_ref, o_ref):
    o_ref[...] = (attn_ref[0:1, :] * r0_ref[...]
                  + attn_ref[1:2, :] * r1_ref[...])


def _combine2(r0, r1, attn):
    return pl.pallas_call(
        _combine_body,
        grid=(GRID,),
        in_specs=[
            pl.BlockSpec((BR, C), lambda i: (i, 0)),
            pl.BlockSpec((BR, C), lambda i: (i, 0)),
            pl.BlockSpec((2, C), lambda i: (0, 0)),
        ],
        out_specs=pl.BlockSpec((BR, C), lambda i: (i, 0)),
        out_shape=jax.ShapeDtypeStruct((NPAD, C), F32),
    )(r0, r1, attn)


@functools.lru_cache(maxsize=None)
def _make_sc_edge(epad, as_base, ad_base):
    nch = epad // (NW * EK)
    mesh = plsc.VectorSubcoreMesh(core_axis_name="c", subcore_axis_name="s")
    cp = pltpu.CompilerParams()
    for fld, val in (("needs_layout_passes", False),
                     ("use_tc_tiling_on_sc", False)):
        if fld in pltpu.CompilerParams.__dataclass_fields__:
            cp = dataclasses.replace(cp, **{fld: val})

    @functools.partial(
        pl.kernel,
        compiler_params=cp,
        out_type=jax.ShapeDtypeStruct((2, NACC, AUGW), F32),
        mesh=mesh,
        scratch_types=[
            pltpu.VMEM((EK,), I32),
            pltpu.VMEM((EK,), I32),
            pltpu.VMEM((EK, C), F32),
            pltpu.VMEM((EK, 16), F32),
            pltpu.VMEM((EK, 16), F32),
            pltpu.VMEM((EK * HH,), F32),
            pltpu.VMEM((EK, AUGW), F32),
            pltpu.VMEM((16,), F32),
            pltpu.VMEM_SHARED((NACC, AUGW), F32),
            pltpu.SemaphoreType.DMA,
            pltpu.SemaphoreType.DMA,
            pltpu.SemaphoreType.DMA,
        ],
    )
    def sc_edge(h_hbm, as_hbm, ad_hbm, maxs_hbm, src_hbm, dst_hbm, out_hbm,
                srcv, dstv, rows, asv, adv, sflat, obuf, maxv, acc,
                sem0, sem1, sem2):
        cid = lax.axis_index("c")
        sid = lax.axis_index("s")
        wid = sid * 2 + cid

        pltpu.sync_copy(maxs_hbm, maxv)

        zero16 = jnp.zeros((16,), F32)

        @pl.loop(0, EK)
        def _(j):
            for g in range(AUGW // 16):
                obuf[j, pl.ds(g * 16, 16)] = zero16

        for k in range(NSLICE // EK):
            pltpu.sync_copy(obuf, acc.at[pl.ds(sid * NSLICE + k * EK, EK)])
        _rem = NSLICE % EK
        if _rem:
            pltpu.sync_copy(
                obuf.at[pl.ds(0, _rem)],
                acc.at[pl.ds(sid * NSLICE + (NSLICE // EK) * EK, _rem)])
        plsc.subcore_barrier()

        lane = lax.iota(I32, 16)
        rowpat = lane >> 2
        ascol = (lane & 3) + as_base
        adcol = (lane & 3) + ad_base
        low4 = lane < 4
        s_at = lane & 3

        @pl.loop(0, nch)
        def _(ci):
            base = (wid * nch + ci) * EK
            pltpu.sync_copy(src_hbm.at[pl.ds(base, EK)], srcv)
            pltpu.sync_copy(dst_hbm.at[pl.ds(base, EK)], dstv)
            cp_rows = pltpu.async_copy(h_hbm.at[srcv], rows, sem0)
            cp_as = pltpu.async_copy(as_hbm.at[srcv], asv, sem1)
            cp_ad = pltpu.async_copy(ad_hbm.at[dstv], adv, sem2)
            cp_as.wait()
            cp_ad.wait()
            mvec = maxv[...]

            @pl.loop(0, EK // 4)
            def _(e4):
                ridx = rowpat + e4 * 4
                a_s = plsc.load_gather(asv, [ridx, ascol])
                a_d = plsc.load_gather(adv, [ridx, adcol])
                t = a_s + a_d
                alpha = jnp.where(t >= 0, t, 0.2 * t)
                u = mvec + a_d
                m = jnp.where(u >= 0, u, 0.2 * u)
                sflat[pl.ds(e4 * 16, 16)] = jnp.exp(alpha - m)

            cp_rows.wait()

            @pl.loop(0, EK)
            def _(j):
                jbase = j * HH
                for h in range(HH):
                    w = plsc.load_gather(
                        sflat, [jnp.full((16,), jbase + h, I32)])
                    for g2 in range(2):
                        g = h * 2 + g2
                        obuf[j, pl.ds(g * 16, 16)] = (
                            rows[j, pl.ds(g * 16, 16)] * w)
                svals = plsc.load_gather(sflat, [jbase + s_at])
                obuf[j, pl.ds(C, 16)] = jnp.where(low4, svals, 0.0)

            pltpu.sync_copy(obuf, acc.at[dstv], add=True)

        plsc.subcore_barrier()
        for k in range(NSLICE // EK):
            off = sid * NSLICE + k * EK
            pltpu.sync_copy(acc.at[pl.ds(off, EK)],
                            out_hbm.at[cid, pl.ds(off, EK)])
        if _rem:
            off = sid * NSLICE + (NSLICE // EK) * EK
            pltpu.sync_copy(acc.at[pl.ds(off, _rem)],
                            out_hbm.at[cid, pl.ds(off, _rem)])

    return sc_edge


def _blockdiag(avec):
    eye = jnp.eye(HH, dtype=F32)
    return (avec[:, :, None] * eye[:, None, :]).reshape(C, HH)


def _amat(lp, nt):
    z8 = jnp.zeros((C, 8), F32)
    if nt == 'user':
        cols = [_blockdiag(lp['a_src']['ui']), _blockdiag(lp['a_dst']['iu']), z8]
    elif nt == 'item':
        cols = [_blockdiag(lp['a_src']['iu']), _blockdiag(lp['a_src']['it']),
                _blockdiag(lp['a_dst']['ui']), _blockdiag(lp['a_dst']['ti'])]
    else:
        cols = [_blockdiag(lp['a_src']['ti']), _blockdiag(lp['a_dst']['it']), z8]
    return jnp.concatenate(cols, axis=1)


_COLS = {'ui': (0, 8), 'iu': (0, 4), 'it': (4, 4), 'ti': (0, 12)}


def kernel(x_user, x_item, x_tag, params, ei_ui, ei_iu, ei_it, ei_ti):
    def padn(x):
        return jnp.pad(x.astype(F32), ((0, NPAD - x.shape[0]), (0, 0)))

    edges = {}
    for et, ei in (('ui', ei_ui), ('iu', ei_iu), ('it', ei_it), ('ti', ei_ti)):
        e = ei.shape[1]
        epad = -(-e // (NW * EK)) * (NW * EK)
        src = jnp.concatenate(
            [ei[0].astype(I32), jnp.zeros((epad - e,), I32)])
        dst = jnp.concatenate(
            [ei[1].astype(I32), jnp.full((epad - e,), NACC - 1, I32)])
        edges[et] = (src, dst, epad)

    r = {'user': (padn(x_user),), 'item': (padn(x_item),),
         'tag': (padn(x_tag),)}
    attn_item = None
    chain = jnp.zeros((1,), F32)

    for lp in params:
        h, aux, mx = {}, {}, {}
        for nt in _NT:
            w = lp['proj_W'][nt].astype(F32)
            b = lp['proj_b'][nt].astype(F32).reshape(1, C)
            am = _amat(lp, nt)
            if len(r[nt]) == 2:
                h[nt], aux[nt], mx[nt] = _proj2(
                    r[nt][0], r[nt][1], attn_item, w, b, am)
            else:
                h[nt], aux[nt], mx[nt] = _proj1(r[nt][0], w, b, am)

        part = {}
        for et, (st, dt) in _EM.items():
            src, dst, epad = edges[et]
            as_base, ad_base = _COLS[et]
            maxs = jnp.tile(mx[st][0, as_base:as_base + 4], 4)
            maxs, _ = lax.optimization_barrier((maxs, chain))
            sc_edge = _make_sc_edge(epad, as_base, ad_base)
            p = sc_edge(h[st], aux[st], aux[dt], maxs, src, dst)
            chain = p[0, 0, :1]
            part[et] = jnp.pad(p, ((0, 0), (0, NPAD - NACC), (0, 0)))

        kw = lp['k_W'].astype(F32)
        kb = lp['k_b'].astype(F32).reshape(1, C)
        q = lp['q'].astype(F32).reshape(1, C)
        r_user = _post_simple(part['iu'])
        r_tag = _post_simple(part['it'])
        r_ui, sc_ui = _post_scored(part['ui'], kw, kb, q)
        r_ti, sc_ti = _post_scored(part['ti'], kw, kb, q)
        scores = jnp.stack([sc_ui[0, 0], sc_ti[0, 0]]) / NNODE
        attn = jax.nn.softmax(scores)
        attn_item = jnp.broadcast_to(attn[:, None], (2, C))
        r = {'user': (r_user,), 'item': (r_ui, r_ti), 'tag': (r_tag,)}

    out_user = r['user'][0][:NNODE]
    out_tag = r['tag'][0][:NNODE]
    out_item = _combine2(r['item'][0], r['item'][1], attn_item)[:NNODE]
    return out_user, out_item, out_tag
